# Initial kernel scaffold; baseline (speedup 1.0000x reference)
import jax
import jax.numpy as jnp
from jax.experimental import pallas as pl


def kernel(x, edge_index, batch, W1, b1, W2, b2, W3, b3, Wl, bl):
    raise NotImplementedError("write your pallas kernel here")


# speedup vs baseline: 10.2100x; 10.2100x over previous
import functools

import jax
import jax.numpy as jnp
from jax import lax
from jax.experimental import pallas as pl
from jax.experimental.pallas import tpu as pltpu
from jax.experimental.pallas import tpu_sc as plsc

G = 128

NCORES = 2
NSUB = 16
NW = NCORES * NSUB
CH = 128


def _cdiv(a, b):
    return (a + b - 1) // b


def _make_deg_kernel(nc, dr, dpt):
    mesh = plsc.VectorSubcoreMesh(core_axis_name="c", subcore_axis_name="s")

    @functools.partial(
        pl.kernel,
        out_type=jax.ShapeDtypeStruct((NCORES, dr), jnp.float32),
        mesh=mesh,
        scratch_types=[
            pltpu.VMEM_SHARED((dr,), jnp.float32),
            pltpu.VMEM((nc, CH), jnp.int32),
            pltpu.VMEM((CH,), jnp.float32),
        ],
    )
    def deg_kernel(dsts, ones_h, zeros_h, out, acc, dst_idx, ones_v):
        cid = lax.axis_index("c")
        sid = lax.axis_index("s")
        wid = cid * NSUB + sid
        pltpu.sync_copy(zeros_h, acc.at[pl.ds(sid * dpt, dpt)])
        pltpu.sync_copy(ones_h, ones_v)
        pltpu.sync_copy(dsts.at[wid], dst_idx)
        plsc.subcore_barrier()

        @pl.loop(0, nc)
        def _(j):
            pltpu.sync_copy(ones_v, acc.at[dst_idx.at[j]], add=True)

        plsc.subcore_barrier()
        pltpu.sync_copy(acc.at[pl.ds(sid * dpt, dpt)],
                        out.at[cid, pl.ds(sid * dpt, dpt)])

    return deg_kernel


def _make_prop_kernel(n, h, nc, ar, rpt):
    mesh = plsc.VectorSubcoreMesh(core_axis_name="c", subcore_axis_name="s")

    @functools.partial(
        pl.kernel,
        out_type=jax.ShapeDtypeStruct((NCORES, ar, h), jnp.float32),
        mesh=mesh,
        scratch_types=[
            pltpu.VMEM_SHARED((ar, h), jnp.float32),
            pltpu.VMEM((nc, CH), jnp.int32),
            pltpu.VMEM((nc, CH), jnp.int32),
            pltpu.VMEM((CH, h), jnp.float32),
        ],
    )
    def prop_kernel(srcs, dsts, y, zeros_h, out, acc, src_idx, dst_idx, rows):
        cid = lax.axis_index("c")
        sid = lax.axis_index("s")
        wid = cid * NSUB + sid
        pltpu.sync_copy(zeros_h, acc.at[pl.ds(sid * rpt, rpt)])
        pltpu.sync_copy(srcs.at[wid], src_idx)
        pltpu.sync_copy(dsts.at[wid], dst_idx)
        plsc.subcore_barrier()

        @pl.loop(0, nc)
        def _(j):
            pltpu.sync_copy(y.at[src_idx.at[j]], rows)
            pltpu.sync_copy(rows, acc.at[dst_idx.at[j]], add=True)

        plsc.subcore_barrier()
        pltpu.sync_copy(acc.at[pl.ds(sid * rpt, rpt)],
                        out.at[cid, pl.ds(sid * rpt, rpt)])

    return prop_kernel


def _mm_scale_body(x_ref, w_ref, deg_ref, y_ref):
    d = deg_ref[:, 0:1] + deg_ref[:, 1:2] + 1.0
    dinv = lax.rsqrt(d)
    xw = jnp.dot(x_ref[...], w_ref[...], preferred_element_type=jnp.float32)
    y_ref[...] = xw * dinv


def _layer_body(p_ref, y_ref, deg_ref, b_ref, w_ref, o_ref):
    d = deg_ref[:, 0:1] + deg_ref[:, 1:2] + 1.0
    dinv = lax.rsqrt(d)
    s = p_ref[0] + p_ref[1] + y_ref[...]
    hh = jnp.maximum(s * dinv + b_ref[...], 0.0)
    o_ref[...] = jnp.dot(hh, w_ref[...], preferred_element_type=jnp.float32) * dinv


def _final_body(nblk, rblk, p_ref, y_ref, deg_ref, b_ref, batch_ref, wl_ref,
                bl_ref, o_ref, pool_acc, cnt_acc):
    i = pl.program_id(0)

    @pl.when(i == 0)
    def _():
        pool_acc[...] = jnp.zeros_like(pool_acc)
        cnt_acc[...] = jnp.zeros_like(cnt_acc)

    d = deg_ref[:, 0:1] + deg_ref[:, 1:2] + 1.0
    dinv = lax.rsqrt(d)
    s = p_ref[0] + p_ref[1] + y_ref[...]
    hh = jnp.maximum(s * dinv + b_ref[...], 0.0)
    seg = (batch_ref[...] == lax.broadcasted_iota(jnp.int32, (rblk, G), 1))
    seg = seg.astype(jnp.float32)
    dn = (((0,), (0,)), ((), ()))
    pool_acc[...] += lax.dot_general(seg, hh, dn,
                                     preferred_element_type=jnp.float32)
    cnt_acc[...] += lax.dot_general(seg, jnp.ones((rblk, G), jnp.float32), dn,
                                    preferred_element_type=jnp.float32)

    @pl.when(i == nblk - 1)
    def _():
        hdim = pool_acc.shape[1]
        pooled = pool_acc[...] / jnp.maximum(cnt_acc[:, :hdim], 1.0)
        o_ref[...] = (jnp.dot(pooled, wl_ref[...],
                              preferred_element_type=jnp.float32) + bl_ref[...])


def kernel(x, edge_index, batch, W1, b1, W2, b2, W3, b3, Wl, bl):
    n, f_in = x.shape
    h0 = W1.shape[1]
    c = Wl.shape[1]
    e = edge_index.shape[1]

    h = 128
    hp = h - h0
    W1 = jnp.pad(W1, ((0, 0), (0, hp)))
    W2 = jnp.pad(W2, ((0, h - W2.shape[0]), (0, hp)))
    W3 = jnp.pad(W3, ((0, h - W3.shape[0]), (0, hp)))
    Wl = jnp.pad(Wl, ((0, h - Wl.shape[0]), (0, 0)))
    b1 = jnp.pad(b1, (0, hp))
    b2 = jnp.pad(b2, (0, hp))
    b3 = jnp.pad(b3, (0, hp))

    nc = _cdiv(e, NW * CH)
    e_pad = NW * nc * CH
    rpt = _cdiv(n + 1, NSUB)
    rpt = _cdiv(rpt, 8) * 8
    ar = NSUB * rpt
    dpt = _cdiv(n + 1, NSUB)
    dpt = _cdiv(dpt, 16) * 16
    dr = NSUB * dpt

    pad = e_pad - e
    srcs = jnp.concatenate([edge_index[0], jnp.zeros((pad,), jnp.int32)])
    dsts = jnp.concatenate([edge_index[1], jnp.full((pad,), n, jnp.int32)])
    srcs = srcs.reshape(NW, nc, CH)
    dsts = dsts.reshape(NW, nc, CH)

    ones_h = jnp.ones((CH,), jnp.float32)
    zeros_d = jnp.zeros((dpt,), jnp.float32)
    zeros_p = jnp.zeros((rpt, h), jnp.float32)
    batch2d = batch.reshape(n, 1)
    b1r = b1.reshape(1, h)
    b2r = b2.reshape(1, h)
    b3r = b3.reshape(1, h)
    blr = bl.reshape(1, c)

    deg_kernel = _make_deg_kernel(nc, dr, dpt)
    prop_kernel = _make_prop_kernel(n, h, nc, ar, rpt)

    rblk = 2000
    nblk = n // rblk

    def row_spec(width):
        return pl.BlockSpec((rblk, width), lambda i: (i, 0))

    full = lambda shape: pl.BlockSpec(shape, lambda i: (0,) * len(shape))
    p_spec = pl.BlockSpec((NCORES, rblk, h), lambda i: (0, i, 0))

    mm_scale = pl.pallas_call(
        _mm_scale_body,
        grid=(nblk,),
        in_specs=[row_spec(f_in), full((f_in, h)), row_spec(2)],
        out_specs=row_spec(h),
        out_shape=jax.ShapeDtypeStruct((n, h), jnp.float32),
    )

    layer = pl.pallas_call(
        _layer_body,
        grid=(nblk,),
        in_specs=[p_spec, row_spec(h), row_spec(2), full((1, h)),
                  full((h, h))],
        out_specs=row_spec(h),
        out_shape=jax.ShapeDtypeStruct((n, h), jnp.float32),
    )

    final = pl.pallas_call(
        functools.partial(_final_body, nblk, rblk),
        grid=(nblk,),
        in_specs=[p_spec, row_spec(h), row_spec(2), full((1, h)),
                  row_spec(1), full((h, c)), full((1, c))],
        out_specs=pl.BlockSpec((G, c), lambda i: (0, 0)),
        out_shape=jax.ShapeDtypeStruct((G, c), jnp.float32),
        scratch_shapes=[pltpu.VMEM((G, h), jnp.float32),
                        pltpu.VMEM((G, G), jnp.float32)],
    )

    deg = deg_kernel(dsts, ones_h, zeros_d)
    deg_t = deg[:, :n].T

    y1 = mm_scale(x, W1, deg_t)
    p1 = prop_kernel(srcs, dsts, y1, zeros_p)
    y2 = layer(p1[:, :n], y1, deg_t, b1r, W2)
    p2 = prop_kernel(srcs, dsts, y2, zeros_p)
    y3 = layer(p2[:, :n], y2, deg_t, b2r, W3)
    p3 = prop_kernel(srcs, dsts, y3, zeros_p)
    out = final(p3[:, :n], y3, deg_t, b3r, batch2d, Wl, blr)
    return out
